# baseline (speedup 1.0000x reference)
import functools

import jax
import jax.numpy as jnp
from jax.experimental import pallas as pl
from jax.experimental.pallas import tpu as pltpu

D = 32
H = 2
DH = D // H
NB = 6
NL = 2
TOPK = 3
NVIS = 128

G = 8

_NEG = -1e30

_BF = jnp.bfloat16


def _dotb(a, b, transpose_b=False):
    if transpose_b:
        dims = (((2,), (2,)), ((0,), (0,)))
    else:
        dims = (((2,), (1,)), ((0,), (0,)))
    return jax.lax.dot_general(a.astype(_BF), b.astype(_BF), dims,
                               preferred_element_type=jnp.float32)


def _proj(h2d, w_ref, b_ref):
    return jnp.dot(h2d.astype(_BF), w_ref[...].astype(_BF),
                   preferred_element_type=jnp.float32) + b_ref[...]


def _topk_softmax_matrix_t(st):
    nm = float(st.shape[1])
    iota = jax.lax.broadcasted_iota(jnp.int32, st.shape, 1).astype(jnp.float32)

    def pick(sc):
        m = jnp.max(sc, axis=1, keepdims=True)
        is_max = sc >= m
        idx = jnp.min(jnp.where(is_max, iota, nm), axis=1, keepdims=True)
        onehot = iota == idx
        return m, onehot

    m1, o1 = pick(st)
    st = jnp.where(o1, _NEG, st)
    m2, o2 = pick(st)
    st = jnp.where(o2, _NEG, st)
    m3, o3 = pick(st)

    e2 = jnp.exp(m2 - m1)
    e3 = jnp.exp(m3 - m1)
    z = 1.0 + e2 + e3
    w1 = 1.0 / z
    w2 = e2 / z
    w3 = e3 / z
    return jnp.where(o1, w1, jnp.where(o2, w2, jnp.where(o3, w3, 0.0)))


def _layer(h, lw_ref, lb_ref, li):
    g, n, _ = h.shape
    h2 = h.reshape(g * n, D)
    h2b = h2.astype(_BF)
    wqkv = jnp.concatenate([lw_ref[li, 0], lw_ref[li, 1], lw_ref[li, 2]],
                           axis=1).astype(_BF)
    bqkv = jnp.concatenate([lb_ref[li, 0], lb_ref[li, 1], lb_ref[li, 2]],
                           axis=-1)
    qkv = (jnp.dot(h2b, wqkv, preferred_element_type=jnp.float32)
           + bqkv).reshape(g, n, 3 * D)
    q = qkv[:, :, :D]
    k = qkv[:, :, D:2 * D]
    v = qkv[:, :, 2 * D:]

    heads = []
    scale = 1.0 / (DH ** 0.5)
    for hh in range(H):
        sl = slice(hh * DH, (hh + 1) * DH)
        qh = q[:, :, sl] * scale
        kh = k[:, :, sl]
        vh = v[:, :, sl]
        st = _dotb(kh, qh, transpose_b=True)
        at = _topk_softmax_matrix_t(st)
        heads.append(jax.lax.dot_general(
            at.astype(_BF), vh.astype(_BF), (((1,), (1,)), ((0,), (0,))),
            preferred_element_type=jnp.float32))
    out = jnp.concatenate(heads, axis=-1).reshape(g * n, D)
    out = jnp.dot(out.astype(_BF), lw_ref[li, 3].astype(_BF),
                  preferred_element_type=jnp.float32) + lb_ref[li, 3]
    return out.reshape(g, n, D)


def _ln(x, gamma, beta):
    mu = jnp.mean(x, axis=-1, keepdims=True)
    var = jnp.var(x, axis=-1, keepdims=True)
    return (x - mu) / jnp.sqrt(var + 1e-5) * gamma + beta


def _fwd_kernel(x_ref, w1_ref, b1_ref, w2_ref, b2_ref, sp_ref,
                lw_ref, lb_ref, lng_ref, lnb_ref, hw_ref, hb_ref, out_ref):
    xb = x_ref[...]
    safe = jnp.maximum(xb, 1e-6)
    w1 = w1_ref[...].astype(_BF).astype(jnp.float32)
    xbt = xb.astype(_BF).astype(jnp.float32)
    lgt = jnp.log(safe).astype(_BF).astype(jnp.float32)
    sqt = jnp.sqrt(safe).astype(_BF).astype(jnp.float32)
    h = (xbt[:, :, None] * w1[0][None, None, :]
         + lgt[:, :, None] * w1[1][None, None, :]
         + sqt[:, :, None] * w1[2][None, None, :]
         + b1_ref[...][None, :, :])
    h = h * (jax.lax.erf(h / 1.4142135623730951) + 1.0) * 0.5
    h2 = _proj(h.reshape(G * NVIS, D), w2_ref, b2_ref).reshape(G, NVIS, D)
    h2 = h2 + sp_ref[...][None, :, :]

    for li in range(NL):
        attn = _layer(h2, lw_ref, lb_ref, li)
        h2 = _ln(h2 + attn, lng_ref[li], lnb_ref[li])

    basis = _proj(h2.reshape(G * NVIS, D), hw_ref, hb_ref)
    out_ref[...] = basis.reshape(G, NVIS, NB)


@jax.jit
def kernel(x, params):
    B, T, _ = x.shape
    bt = B * T
    xr = x.reshape(bt, NVIS)

    w1 = params["in1"]["w"]
    b1 = params["in1"]["b"].reshape(1, D)
    w2 = params["in2"]["w"]
    b2 = params["in2"]["b"].reshape(1, D)
    sp = params["species"]
    hw = params["head"]["w"]
    hb = params["head"]["b"].reshape(1, NB)
    lw = jnp.stack([jnp.stack([lp["q"]["w"], lp["k"]["w"], lp["v"]["w"],
                               lp["out"]["w"]]) for lp in params["layers"]])
    lb = jnp.stack([jnp.stack([lp["q"]["b"], lp["k"]["b"], lp["v"]["b"],
                               lp["out"]["b"]]) for lp in params["layers"]])
    lng = jnp.stack([lp["ln_g"] for lp in params["layers"]])
    lnb = jnp.stack([lp["ln_b"] for lp in params["layers"]])

    grid = (bt // G,)
    rep2 = lambda i: (0, 0)
    rep3 = lambda i: (0, 0, 0)
    rep4 = lambda i: (0, 0, 0, 0)
    out = pl.pallas_call(
        _fwd_kernel,
        grid=grid,
        in_specs=[
            pl.BlockSpec((G, NVIS), lambda i: (i, 0)),
            pl.BlockSpec((3, D), rep2),
            pl.BlockSpec((1, D), rep2),
            pl.BlockSpec((D, D), rep2),
            pl.BlockSpec((1, D), rep2),
            pl.BlockSpec((NVIS, D), rep2),
            pl.BlockSpec((NL, 4, D, D), rep4),
            pl.BlockSpec((NL, 4, D), rep3),
            pl.BlockSpec((NL, D), rep2),
            pl.BlockSpec((NL, D), rep2),
            pl.BlockSpec((D, NB), rep2),
            pl.BlockSpec((1, NB), rep2),
        ],
        out_specs=pl.BlockSpec((G, NVIS, NB), lambda i: (i, 0, 0)),
        out_shape=jax.ShapeDtypeStruct((bt, NVIS, NB), jnp.float32),
        compiler_params=pltpu.CompilerParams(
            dimension_semantics=("parallel",),
            fuse_transposed_lhs_in_matmul=True),
    )(xr, w1, b1, w2, b2, sp, lw, lb, lng, lnb, hw, hb)
    return out.reshape(B, T, NVIS, NB)


# speedup vs baseline: 1.0251x; 1.0251x over previous
import functools

import jax
import jax.numpy as jnp
from jax.experimental import pallas as pl
from jax.experimental.pallas import tpu as pltpu

D = 32
H = 2
DH = D // H
NB = 6
NL = 2
TOPK = 3
NVIS = 128

G = 8

_NEG = -1e30

_BF = jnp.bfloat16


def _dotb(a, b, transpose_b=False):
    if transpose_b:
        dims = (((2,), (2,)), ((0,), (0,)))
    else:
        dims = (((2,), (1,)), ((0,), (0,)))
    return jax.lax.dot_general(a.astype(_BF), b.astype(_BF), dims,
                               preferred_element_type=jnp.float32)


def _proj(h2d, w_ref, b_ref):
    return jnp.dot(h2d.astype(_BF), w_ref[...].astype(_BF),
                   preferred_element_type=jnp.float32) + b_ref[...]


def _topk_softmax_matrix_t(st):
    nm = float(st.shape[1])
    iota = jax.lax.broadcasted_iota(jnp.int32, st.shape, 1).astype(jnp.float32)

    def pick(sc):
        m = jnp.max(sc, axis=1, keepdims=True)
        is_max = sc >= m
        idx = jnp.min(jnp.where(is_max, iota, nm), axis=1, keepdims=True)
        onehot = iota == idx
        return m, onehot

    m1, o1 = pick(st)
    st = jnp.where(o1, _NEG, st)
    m2, o2 = pick(st)
    st = jnp.where(o2, _NEG, st)
    m3, o3 = pick(st)

    e2 = jnp.exp(m2 - m1)
    e3 = jnp.exp(m3 - m1)
    z = 1.0 + e2 + e3
    w1 = 1.0 / z
    w2 = e2 / z
    w3 = e3 / z
    return jnp.where(o1, w1, jnp.where(o2, w2, jnp.where(o3, w3, 0.0)))


def _layer(h, lw_ref, lb_ref, li):
    g, n, _ = h.shape
    h2 = h.reshape(g * n, D)
    h2b = h2.astype(_BF)
    wqkv = jnp.concatenate([lw_ref[li, 0], lw_ref[li, 1], lw_ref[li, 2]],
                           axis=1).astype(_BF)
    bqkv = jnp.concatenate([lb_ref[li, 0], lb_ref[li, 1], lb_ref[li, 2]],
                           axis=-1)
    qkv = (jnp.dot(h2b, wqkv, preferred_element_type=jnp.float32)
           + bqkv).reshape(g, n, 3 * D)
    q = qkv[:, :, :D]
    k = qkv[:, :, D:2 * D]
    v = qkv[:, :, 2 * D:]

    heads = []
    scale = 1.0 / (DH ** 0.5)
    for hh in range(H):
        sl = slice(hh * DH, (hh + 1) * DH)
        qh = q[:, :, sl] * scale
        kh = k[:, :, sl]
        vh = v[:, :, sl]
        st = _dotb(kh, qh, transpose_b=True)
        at = _topk_softmax_matrix_t(st)
        heads.append(jax.lax.dot_general(
            at.astype(_BF), vh.astype(_BF), (((1,), (1,)), ((0,), (0,))),
            preferred_element_type=jnp.float32))
    out = jnp.concatenate(heads, axis=-1).reshape(g * n, D)
    out = jnp.dot(out.astype(_BF), lw_ref[li, 3].astype(_BF),
                  preferred_element_type=jnp.float32) + lb_ref[li, 3]
    return out.reshape(g, n, D)


def _ln(x, gamma, beta):
    mu = jnp.mean(x, axis=-1, keepdims=True)
    var = jnp.var(x, axis=-1, keepdims=True)
    return (x - mu) / jnp.sqrt(var + 1e-5) * gamma + beta


def _fwd_kernel(x_ref, w1_ref, b1_ref, w2_ref, b2_ref, sp_ref,
                lw_ref, lb_ref, lng_ref, lnb_ref, hw_ref, hb_ref, out_ref):
    xb = x_ref[...]
    safe = jnp.maximum(xb, 1e-6)
    w1 = w1_ref[...].astype(_BF).astype(jnp.float32)
    xbt = xb.astype(_BF).astype(jnp.float32)
    lgt = jnp.log(safe).astype(_BF).astype(jnp.float32)
    sqt = jnp.sqrt(safe).astype(_BF).astype(jnp.float32)
    h = (xbt[:, :, None] * w1[0][None, None, :]
         + lgt[:, :, None] * w1[1][None, None, :]
         + sqt[:, :, None] * w1[2][None, None, :]
         + b1_ref[...][None, :, :])
    h = h * (jax.lax.erf(h / 1.4142135623730951) + 1.0) * 0.5
    h2 = _proj(h.reshape(G * NVIS, D), w2_ref, b2_ref).reshape(G, NVIS, D)
    h2 = h2 + sp_ref[...][None, :, :]

    for li in range(NL):
        attn = _layer(h2, lw_ref, lb_ref, li)
        h2 = _ln(h2 + attn, lng_ref[li], lnb_ref[li])

    basis = _proj(h2.reshape(G * NVIS, D), hw_ref, hb_ref)
    out_ref[...] = basis.reshape(G, NVIS, NB)


@jax.jit
def kernel(x, params):
    B, T, _ = x.shape
    bt = B * T
    xr = x.reshape(bt, NVIS)

    w1 = params["in1"]["w"]
    b1 = params["in1"]["b"].reshape(1, D)
    w2 = params["in2"]["w"]
    b2 = params["in2"]["b"].reshape(1, D)
    sp = params["species"]
    hw = params["head"]["w"]
    hb = params["head"]["b"].reshape(1, NB)
    lw = jnp.stack([jnp.stack([lp["q"]["w"], lp["k"]["w"], lp["v"]["w"],
                               lp["out"]["w"]]) for lp in params["layers"]])
    lb = jnp.stack([jnp.stack([lp["q"]["b"], lp["k"]["b"], lp["v"]["b"],
                               lp["out"]["b"]]) for lp in params["layers"]])
    lng = jnp.stack([lp["ln_g"] for lp in params["layers"]])
    lnb = jnp.stack([lp["ln_b"] for lp in params["layers"]])

    grid = (bt // G,)
    rep2 = lambda i: (0, 0)
    rep3 = lambda i: (0, 0, 0)
    rep4 = lambda i: (0, 0, 0, 0)
    out = pl.pallas_call(
        _fwd_kernel,
        grid=grid,
        in_specs=[
            pl.BlockSpec((G, NVIS), lambda i: (i, 0)),
            pl.BlockSpec((3, D), rep2),
            pl.BlockSpec((1, D), rep2),
            pl.BlockSpec((D, D), rep2),
            pl.BlockSpec((1, D), rep2),
            pl.BlockSpec((NVIS, D), rep2),
            pl.BlockSpec((NL, 4, D, D), rep4),
            pl.BlockSpec((NL, 4, D), rep3),
            pl.BlockSpec((NL, D), rep2),
            pl.BlockSpec((NL, D), rep2),
            pl.BlockSpec((D, NB), rep2),
            pl.BlockSpec((1, NB), rep2),
        ],
        out_specs=pl.BlockSpec((G, NVIS, NB), lambda i: (i, 0, 0)),
        out_shape=jax.ShapeDtypeStruct((bt, NVIS, NB), jnp.float32),
        compiler_params=pltpu.CompilerParams(
            dimension_semantics=("parallel",)),
    )(xr, w1, b1, w2, b2, sp, lw, lb, lng, lnb, hw, hb)
    return out.reshape(B, T, NVIS, NB)
